# baseline (speedup 1.0000x reference)
import jax
import jax.numpy as jnp
from jax import lax
from jax.experimental import pallas as pl
from jax.experimental.pallas import tpu as pltpu
from jax.experimental.pallas import tpu_sc as plsc

_TIMESTEPS = 1000
_N = 16384
_TABLE = 1001
_RN_MAGIC = 12582912.0

_NC, _NS, _L = 2, 16, 16
_NW = _NC * _NS
_CHUNK = _N // _NW


def _gather_body(t_hbm, gamma_hbm, out_hbm, t_v, gamma_v, out_v, sem_g, sem_t, sem_o):
    wid = lax.axis_index("s") * _NC + lax.axis_index("c")
    base = wid * _CHUNK
    nv = _CHUNK // _L
    half = nv // 2
    cp_g = pltpu.async_copy(gamma_hbm, gamma_v, sem_g)
    cp_t = pltpu.async_copy(t_hbm.at[pl.ds(base, _CHUNK)], t_v, sem_t)
    cp_t.wait()
    cp_g.wait()

    def _lookup(i):
        y = t_v[pl.ds(i * _L, _L)] * float(_TIMESTEPS)
        idx = ((y + _RN_MAGIC) - _RN_MAGIC).astype(jnp.int32)
        out_v[pl.ds(i * _L, _L)] = plsc.load_gather(gamma_v, [idx])

    half = nv // 2
    for i in range(half):
        _lookup(i)
    cp_o1 = pltpu.async_copy(
        out_v.at[pl.ds(0, half * _L)], out_hbm.at[pl.ds(base, half * _L)], sem_o
    )
    for i in range(half, nv):
        _lookup(i)
    cp_o2 = pltpu.async_copy(
        out_v.at[pl.ds(half * _L, half * _L)],
        out_hbm.at[pl.ds(base + half * _L, half * _L)],
        sem_o,
    )
    cp_o1.wait()
    cp_o2.wait()


@jax.jit
def kernel(t, gamma):
    run = pl.kernel(
        _gather_body,
        out_type=jax.ShapeDtypeStruct((_N,), jnp.float32),
        mesh=plsc.VectorSubcoreMesh(
            core_axis_name="c", subcore_axis_name="s", num_cores=_NC
        ),
        scratch_types=[
            pltpu.VMEM((_CHUNK,), jnp.float32),
            pltpu.VMEM((_TABLE,), jnp.float32),
            pltpu.VMEM((_CHUNK,), jnp.float32),
            pltpu.SemaphoreType.DMA,
            pltpu.SemaphoreType.DMA,
            pltpu.SemaphoreType.DMA,
        ],
        compiler_params=pltpu.CompilerParams(needs_layout_passes=False),
    )
    return run(t, gamma)


# speedup vs baseline: 1.0578x; 1.0578x over previous
import jax
import jax.numpy as jnp
from jax import lax
from jax.experimental import pallas as pl
from jax.experimental.pallas import tpu as pltpu
from jax.experimental.pallas import tpu_sc as plsc

_TIMESTEPS = 1000
_N = 16384
_TABLE = 1001
_RN_MAGIC = 12582912.0

_NC, _NS, _L = 1, 16, 16
_NW = _NC * _NS
_CHUNK = _N // _NW


def _gather_body(t_hbm, gamma_hbm, out_hbm, t_v, gamma_v, out_v, sem_g, sem_t, sem_o):
    wid = lax.axis_index("s") * _NC + lax.axis_index("c")
    base = wid * _CHUNK
    nv = _CHUNK // _L
    half = nv // 2
    cp_g = pltpu.async_copy(gamma_hbm, gamma_v, sem_g)
    cp_t = pltpu.async_copy(t_hbm.at[pl.ds(base, _CHUNK)], t_v, sem_t)
    cp_t.wait()
    cp_g.wait()

    def _lookup(i):
        y = t_v[pl.ds(i * _L, _L)] * float(_TIMESTEPS)
        idx = ((y + _RN_MAGIC) - _RN_MAGIC).astype(jnp.int32)
        out_v[pl.ds(i * _L, _L)] = plsc.load_gather(gamma_v, [idx])

    half = nv // 2
    for i in range(half):
        _lookup(i)
    cp_o1 = pltpu.async_copy(
        out_v.at[pl.ds(0, half * _L)], out_hbm.at[pl.ds(base, half * _L)], sem_o
    )
    for i in range(half, nv):
        _lookup(i)
    cp_o2 = pltpu.async_copy(
        out_v.at[pl.ds(half * _L, half * _L)],
        out_hbm.at[pl.ds(base + half * _L, half * _L)],
        sem_o,
    )
    cp_o1.wait()
    cp_o2.wait()


@jax.jit
def kernel(t, gamma):
    run = pl.kernel(
        _gather_body,
        out_type=jax.ShapeDtypeStruct((_N,), jnp.float32),
        mesh=plsc.VectorSubcoreMesh(
            core_axis_name="c", subcore_axis_name="s", num_cores=_NC
        ),
        scratch_types=[
            pltpu.VMEM((_CHUNK,), jnp.float32),
            pltpu.VMEM((_TABLE,), jnp.float32),
            pltpu.VMEM((_CHUNK,), jnp.float32),
            pltpu.SemaphoreType.DMA,
            pltpu.SemaphoreType.DMA,
            pltpu.SemaphoreType.DMA,
        ],
        compiler_params=pltpu.CompilerParams(needs_layout_passes=False),
    )
    return run(t, gamma)


# speedup vs baseline: 1.0991x; 1.0390x over previous
import jax
import jax.numpy as jnp
from jax import lax
from jax.experimental import pallas as pl
from jax.experimental.pallas import tpu as pltpu
from jax.experimental.pallas import tpu_sc as plsc

_TIMESTEPS = 1000
_N = 16384
_TABLE = 1001
_RN_MAGIC = 12582912.0

_NC, _NS, _L = 1, 16, 16
_NW = _NC * _NS
_CHUNK = _N // _NW


def _gather_body(t_hbm, gamma_hbm, out_hbm, t_v, gamma_v, out_v, sem_g, sem_t, sem_o):
    wid = lax.axis_index("s") * _NC + lax.axis_index("c")
    base = wid * _CHUNK
    nv = _CHUNK // _L
    cp_g = pltpu.async_copy(gamma_hbm, gamma_v, sem_g)
    cp_t = pltpu.async_copy(t_hbm.at[pl.ds(base, _CHUNK)], t_v, sem_t)
    cp_t.wait()
    cp_g.wait()

    def _lookup(i):
        y = t_v[pl.ds(i * _L, _L)] * float(_TIMESTEPS)
        idx = ((y + _RN_MAGIC) - _RN_MAGIC).astype(jnp.int32)
        out_v[pl.ds(i * _L, _L)] = plsc.load_gather(gamma_v, [idx])

    half = nv // 2

    @plsc.parallel_loop(0, half, step=1, unroll=4)
    def _first(i):
        _lookup(i)

    cp_o1 = pltpu.async_copy(
        out_v.at[pl.ds(0, half * _L)], out_hbm.at[pl.ds(base, half * _L)], sem_o
    )

    @plsc.parallel_loop(half, nv, step=1, unroll=4)
    def _second(i):
        _lookup(i)
    cp_o2 = pltpu.async_copy(
        out_v.at[pl.ds(half * _L, half * _L)],
        out_hbm.at[pl.ds(base + half * _L, half * _L)],
        sem_o,
    )
    cp_o1.wait()
    cp_o2.wait()


@jax.jit
def kernel(t, gamma):
    run = pl.kernel(
        _gather_body,
        out_type=jax.ShapeDtypeStruct((_N,), jnp.float32),
        mesh=plsc.VectorSubcoreMesh(
            core_axis_name="c", subcore_axis_name="s", num_cores=_NC
        ),
        scratch_types=[
            pltpu.VMEM((_CHUNK,), jnp.float32),
            pltpu.VMEM((_TABLE,), jnp.float32),
            pltpu.VMEM((_CHUNK,), jnp.float32),
            pltpu.SemaphoreType.DMA,
            pltpu.SemaphoreType.DMA,
            pltpu.SemaphoreType.DMA,
        ],
        compiler_params=pltpu.CompilerParams(needs_layout_passes=False),
    )
    return run(t, gamma)
